# baseline (speedup 1.0000x reference)
import jax
import jax.numpy as jnp
from jax.experimental import pallas as pl
from jax.experimental.pallas import tpu as pltpu

_TILE = 512


def _dot(a, b, dims, prec=jax.lax.Precision.HIGHEST):
    return jax.lax.dot_general(
        a, b, (dims, ((), ())),
        precision=prec,
        preferred_element_type=jnp.float32,
    )


def _gat_kernel(x_ref, xt_ref, adj_ref, wf_ref, attsrc_ref, attdst_ref,
                bias_ref, out_ref):
    h_all = _dot(x_ref[...], wf_ref[...], (((1,), (0,))))
    h_tile = _dot(xt_ref[...], wf_ref[...], (((1,), (0,))))
    mask = adj_ref[...] > 0.5
    n_heads, out_ch = attsrc_ref.shape
    for h in range(n_heads):
        hh = h_all[:, h * out_ch:(h + 1) * out_ch]
        hht = h_tile[:, h * out_ch:(h + 1) * out_ch]
        src_row = attsrc_ref[h:h + 1, :]
        dst_row = attdst_ref[h:h + 1, :]
        a_s = _dot(hh, src_row, (((1,), (1,))))
        a_d = _dot(dst_row, hht, (((1,), (1,))))
        e = a_s + a_d
        e = jnp.maximum(e, 0.2 * e)
        p = jnp.where(mask, jnp.exp(e), 0.0)
        denom = jnp.sum(p, axis=0, keepdims=True)
        rec = jnp.transpose(1.0 / (denom + 1e-16))
        num = _dot(p, hh, (((0,), (0,))), jax.lax.Precision.DEFAULT)
        out_ref[:, h * out_ch:(h + 1) * out_ch] = (
            num * rec + bias_ref[:, h * out_ch:(h + 1) * out_ch])


def kernel(x, adj, W, att_src, att_dst, bias):
    n, d_in = x.shape
    heads, out_ch = att_src.shape
    wf = W.reshape(d_in, heads * out_ch)
    bias2 = bias.reshape(1, heads * out_ch)
    return pl.pallas_call(
        _gat_kernel,
        grid=(n // _TILE,),
        in_specs=[
            pl.BlockSpec((n, d_in), lambda j: (0, 0)),
            pl.BlockSpec((_TILE, d_in), lambda j: (j, 0)),
            pl.BlockSpec((n, _TILE), lambda j: (0, j)),
            pl.BlockSpec((d_in, heads * out_ch), lambda j: (0, 0)),
            pl.BlockSpec((heads, out_ch), lambda j: (0, 0)),
            pl.BlockSpec((heads, out_ch), lambda j: (0, 0)),
            pl.BlockSpec((1, heads * out_ch), lambda j: (0, 0)),
        ],
        out_specs=pl.BlockSpec((_TILE, heads * out_ch), lambda j: (j, 0)),
        out_shape=jax.ShapeDtypeStruct((n, heads * out_ch), jnp.float32),
        compiler_params=pltpu.CompilerParams(
            dimension_semantics=("parallel",)),
    )(x, x, adj, wf, att_src, att_dst, bias2)


# speedup vs baseline: 1.3035x; 1.3035x over previous
import jax
import jax.numpy as jnp
from jax.experimental import pallas as pl
from jax.experimental.pallas import tpu as pltpu

_CHUNK = 256


def _dot(a, b, dims, prec=jax.lax.Precision.HIGHEST):
    return jax.lax.dot_general(
        a, b, (dims, ((), ())),
        precision=prec,
        preferred_element_type=jnp.float32,
    )


def _gat_kernel(x_ref, adj_hbm, wf_ref, attsrc_ref, attdst_ref, bias_ref,
                out_ref, buf, sem):
    n = x_ref.shape[0]
    n_chunks = n // _CHUNK
    n_heads, out_ch = attsrc_ref.shape

    def copy(c, slot):
        return pltpu.make_async_copy(
            adj_hbm.at[:, pl.ds(c * _CHUNK, _CHUNK)], buf.at[slot], sem.at[slot])

    copy(0, 0).start()
    h_all = _dot(x_ref[...], wf_ref[...], (((1,), (0,))))
    heads_hh = [h_all[:, h * out_ch:(h + 1) * out_ch] for h in range(n_heads)]
    a_s = [_dot(heads_hh[h], attsrc_ref[h:h + 1, :], (((1,), (1,))))
           for h in range(n_heads)]
    a_dt = [_dot(attdst_ref[h:h + 1, :], heads_hh[h], (((1,), (1,))))
            for h in range(n_heads)]
    for c in range(n_chunks):
        if c + 1 < n_chunks:
            copy(c + 1, (c + 1) % 2).start()
        copy(c, c % 2).wait()
        mask = buf[c % 2] > 0.5
        for h in range(n_heads):
            hh = heads_hh[h]
            e = (a_s[h]
                 + a_dt[h][:, c * _CHUNK:(c + 1) * _CHUNK])
            e = jnp.maximum(e, 0.2 * e)
            p = jnp.where(mask, jnp.exp(e), 0.0)
            denom = jnp.sum(p, axis=0, keepdims=True)
            rec = jnp.transpose(1.0 / (denom + 1e-16))
            num = _dot(p, hh, (((0,), (0,))),
                       jax.lax.Precision.DEFAULT)
            out_ref[pl.ds(c * _CHUNK, _CHUNK),
                    h * out_ch:(h + 1) * out_ch] = (
                num * rec + bias_ref[:, h * out_ch:(h + 1) * out_ch])


def kernel(x, adj, W, att_src, att_dst, bias):
    n, d_in = x.shape
    heads, out_ch = att_src.shape
    wf = W.reshape(d_in, heads * out_ch)
    bias2 = bias.reshape(1, heads * out_ch)
    return pl.pallas_call(
        _gat_kernel,
        in_specs=[
            pl.BlockSpec((n, d_in), lambda: (0, 0)),
            pl.BlockSpec(memory_space=pltpu.MemorySpace.HBM),
            pl.BlockSpec((d_in, heads * out_ch), lambda: (0, 0)),
            pl.BlockSpec((heads, out_ch), lambda: (0, 0)),
            pl.BlockSpec((heads, out_ch), lambda: (0, 0)),
            pl.BlockSpec((1, heads * out_ch), lambda: (0, 0)),
        ],
        out_specs=pl.BlockSpec((n, heads * out_ch), lambda: (0, 0)),
        out_shape=jax.ShapeDtypeStruct((n, heads * out_ch), jnp.float32),
        scratch_shapes=[
            pltpu.VMEM((2, n, _CHUNK), jnp.float32),
            pltpu.SemaphoreType.DMA((2,)),
        ],
    )(x, adj, wf, att_src, att_dst, bias2)


# speedup vs baseline: 1.4985x; 1.1496x over previous
import jax
import jax.numpy as jnp
from jax.experimental import pallas as pl


def _dot(a, b, dims, prec=jax.lax.Precision.HIGHEST):
    return jax.lax.dot_general(
        a, b, (dims, ((), ())),
        precision=prec,
        preferred_element_type=jnp.float32,
    )


def _gat_kernel(x_ref, adj_ref, wf_ref, attsrc_ref, attdst_ref, bias_ref,
                out_ref):
    n = x_ref.shape[0]
    h_all = _dot(x_ref[...], wf_ref[...], (((1,), (0,))))
    mask = adj_ref[...] > 0.5
    ones_col = jnp.ones((n, 1), jnp.float32)
    n_heads, out_ch = attsrc_ref.shape
    for h in range(n_heads):
        hh = h_all[:, h * out_ch:(h + 1) * out_ch]
        hh_aug = jnp.concatenate([hh, ones_col], axis=1)
        src_row = attsrc_ref[h:h + 1, :]
        dst_row = attdst_ref[h:h + 1, :]
        a_s = _dot(hh, src_row, (((1,), (1,))))
        a_d = _dot(dst_row, hh, (((1,), (1,))))
        e = a_s + a_d
        e = jnp.maximum(e, 0.2 * e)
        p = jnp.where(mask, jnp.exp(e), 0.0)
        num_aug = _dot(p, hh_aug, (((0,), (0,))),
                       jax.lax.Precision.DEFAULT)
        rec = 1.0 / (num_aug[:, out_ch:out_ch + 1] + 1e-16)
        out_ref[:, h * out_ch:(h + 1) * out_ch] = (
            num_aug[:, :out_ch] * rec
            + bias_ref[:, h * out_ch:(h + 1) * out_ch])


def kernel(x, adj, W, att_src, att_dst, bias):
    n, d_in = x.shape
    heads, out_ch = att_src.shape
    wf = W.reshape(d_in, heads * out_ch)
    bias2 = bias.reshape(1, heads * out_ch)
    return pl.pallas_call(
        _gat_kernel,
        in_specs=[
            pl.BlockSpec((n, d_in), lambda: (0, 0)),
            pl.BlockSpec((n, n), lambda: (0, 0)),
            pl.BlockSpec((d_in, heads * out_ch), lambda: (0, 0)),
            pl.BlockSpec((heads, out_ch), lambda: (0, 0)),
            pl.BlockSpec((heads, out_ch), lambda: (0, 0)),
            pl.BlockSpec((1, heads * out_ch), lambda: (0, 0)),
        ],
        out_specs=pl.BlockSpec((n, heads * out_ch), lambda: (0, 0)),
        out_shape=jax.ShapeDtypeStruct((n, heads * out_ch), jnp.float32),
    )(x, adj, wf, att_src, att_dst, bias2)


# speedup vs baseline: 1.5468x; 1.0323x over previous
import jax
import jax.numpy as jnp
from jax.experimental import pallas as pl


def _dot(a, b, dims, prec=jax.lax.Precision.DEFAULT):
    return jax.lax.dot_general(
        a, b, (dims, ((), ())),
        precision=prec,
        preferred_element_type=jnp.float32,
    )


def _gat_kernel(x_ref, adj_ref, wf_ref, attsrc_ref, attdst_ref, bias_ref,
                out_ref):
    n = x_ref.shape[0]
    h_all = _dot(x_ref[...], wf_ref[...], (((1,), (0,))))
    mask = adj_ref[...] > 0.5
    ones_col = jnp.ones((n, 1), jnp.float32)
    n_heads, out_ch = attsrc_ref.shape
    for h in range(n_heads):
        hh = h_all[:, h * out_ch:(h + 1) * out_ch]
        hh_aug = jnp.concatenate([hh, ones_col], axis=1)
        src_row = attsrc_ref[h:h + 1, :]
        dst_row = attdst_ref[h:h + 1, :]
        a_s = _dot(hh, src_row, (((1,), (1,))))
        a_d = _dot(dst_row, hh, (((1,), (1,))))
        e = a_s + a_d
        e = jnp.maximum(e, 0.2 * e)
        p = jnp.where(mask, jnp.exp(e), 0.0)
        num_aug = _dot(p, hh_aug, (((0,), (0,))),
                       jax.lax.Precision.DEFAULT)
        rec = 1.0 / (num_aug[:, out_ch:out_ch + 1] + 1e-16)
        out_ref[:, h * out_ch:(h + 1) * out_ch] = (
            num_aug[:, :out_ch] * rec
            + bias_ref[:, h * out_ch:(h + 1) * out_ch])


def kernel(x, adj, W, att_src, att_dst, bias):
    n, d_in = x.shape
    heads, out_ch = att_src.shape
    wf = W.reshape(d_in, heads * out_ch)
    bias2 = bias.reshape(1, heads * out_ch)
    return pl.pallas_call(
        _gat_kernel,
        in_specs=[
            pl.BlockSpec((n, d_in), lambda: (0, 0)),
            pl.BlockSpec((n, n), lambda: (0, 0)),
            pl.BlockSpec((d_in, heads * out_ch), lambda: (0, 0)),
            pl.BlockSpec((heads, out_ch), lambda: (0, 0)),
            pl.BlockSpec((heads, out_ch), lambda: (0, 0)),
            pl.BlockSpec((1, heads * out_ch), lambda: (0, 0)),
        ],
        out_specs=pl.BlockSpec((n, heads * out_ch), lambda: (0, 0)),
        out_shape=jax.ShapeDtypeStruct((n, heads * out_ch), jnp.float32),
    )(x, adj, wf, att_src, att_dst, bias2)


# speedup vs baseline: 1.7481x; 1.1301x over previous
import jax
import jax.numpy as jnp
from jax.experimental import pallas as pl


def _dot(a, b, dims, prec=jax.lax.Precision.DEFAULT):
    return jax.lax.dot_general(
        a, b, (dims, ((), ())),
        precision=prec,
        preferred_element_type=jnp.float32,
    )


def _gat_kernel(x_ref, adj_ref, wf_ref, attsrc_ref, attdst_ref, bias_ref,
                out_ref):
    n = x_ref.shape[0]
    h_all = _dot(x_ref[...], wf_ref[...], (((1,), (0,))))
    mask = adj_ref[...] > 0.5
    ones_col = jnp.ones((n, 1), jnp.float32)
    n_heads, out_ch = attsrc_ref.shape
    for h in range(n_heads):
        hh = h_all[:, h * out_ch:(h + 1) * out_ch]
        hh_aug = jnp.concatenate([hh, ones_col], axis=1)
        src_row = attsrc_ref[h:h + 1, :]
        dst_row = attdst_ref[h:h + 1, :]
        a_s = _dot(hh, src_row, (((1,), (1,)))).astype(jnp.bfloat16)
        a_d = _dot(dst_row, hh, (((1,), (1,)))).astype(jnp.bfloat16)
        e = a_s + a_d
        e = jnp.maximum(e, jnp.bfloat16(0.2) * e)
        p = jnp.where(mask, jnp.exp(e), jnp.bfloat16(0.0))
        num_aug = _dot(p, hh_aug, (((0,), (0,))),
                       jax.lax.Precision.DEFAULT)
        rec = 1.0 / (num_aug[:, out_ch:out_ch + 1] + 1e-16)
        out_ref[:, h * out_ch:(h + 1) * out_ch] = (
            num_aug[:, :out_ch] * rec
            + bias_ref[:, h * out_ch:(h + 1) * out_ch])


def kernel(x, adj, W, att_src, att_dst, bias):
    n, d_in = x.shape
    heads, out_ch = att_src.shape
    wf = W.reshape(d_in, heads * out_ch)
    bias2 = bias.reshape(1, heads * out_ch)
    return pl.pallas_call(
        _gat_kernel,
        in_specs=[
            pl.BlockSpec((n, d_in), lambda: (0, 0)),
            pl.BlockSpec((n, n), lambda: (0, 0)),
            pl.BlockSpec((d_in, heads * out_ch), lambda: (0, 0)),
            pl.BlockSpec((heads, out_ch), lambda: (0, 0)),
            pl.BlockSpec((heads, out_ch), lambda: (0, 0)),
            pl.BlockSpec((1, heads * out_ch), lambda: (0, 0)),
        ],
        out_specs=pl.BlockSpec((n, heads * out_ch), lambda: (0, 0)),
        out_shape=jax.ShapeDtypeStruct((n, heads * out_ch), jnp.float32),
    )(x, adj, wf, att_src, att_dst, bias2)
